# baseline (speedup 1.0000x reference)
import math

import jax
import jax.numpy as jnp
from jax.experimental import pallas as pl
from jax.experimental.pallas import tpu as pltpu


_LOG_BASE = math.log(10000.0)
_SEED = 16
_SIZES = (256, 256, 512, 1024, 1024, 1024)
_MAXCH = max(_SIZES)


def _gen_all(o_ref, buf_ref, sem_ref):
    total, cols = o_ref.shape
    j = jax.lax.broadcasted_iota(jnp.int32, (1, cols), 1)
    k = (j // 2).astype(jnp.float32)
    w = jnp.exp(k * jnp.float32(-2.0 * _LOG_BASE / cols))
    phase = jnp.where(j % 2 == 1, jnp.float32(math.pi / 2), jnp.float32(0.0))
    r8 = jax.lax.broadcasted_iota(jnp.int32, (8, cols), 0)
    dmat = jnp.minimum(8 << r8, jnp.int32(_MAXCH // 2)).astype(jnp.float32)
    coefs = 2.0 * jnp.sin(dmat * w + jnp.float32(math.pi / 2))
    cof = {}
    d, ridx = 8, 0
    while d <= _MAXCH // 2:
        cof[d] = coefs[ridx:ridx + 1, :]
        ridx, d = ridx + 1, 2 * d
    r = jax.lax.broadcasted_iota(jnp.int32, (_SEED, cols), 0)
    seed_cur = jnp.sin(r.astype(jnp.float32) * w + phase)

    copies = []
    off = 0
    for c, size in enumerate(_SIZES):
        slot = c % 2
        if c >= 2:
            copies[c - 2].wait()
        buf_ref[slot, 0:_SEED, :] = seed_cur
        n = _SEED
        while n < size:
            d = n // 2
            coef = cof[d]
            prev_lo = buf_ref[slot, 0:d, :]
            prev_hi = buf_ref[slot, d:n, :]
            h1 = coef * prev_hi - prev_lo
            buf_ref[slot, n:n + d, :] = h1
            buf_ref[slot, n + d:2 * n, :] = coef * h1 - prev_hi
            n *= 2
        cp = pltpu.make_async_copy(
            buf_ref.at[slot, pl.ds(0, size), :],
            o_ref.at[pl.ds(off, size), :],
            sem_ref.at[c],
        )
        cp.start()
        copies.append(cp)
        off += size
        if c + 1 < len(_SIZES):
            h = size // 2
            seed_cur = (cof[h] * buf_ref[slot, h:h + _SEED, :]
                        - buf_ref[slot, 0:_SEED, :])
    copies[-2].wait()
    copies[-1].wait()


def kernel(x, encoding):
    seq_len = x.shape[1]
    n_embd = encoding.shape[1]
    return pl.pallas_call(
        _gen_all,
        out_specs=pl.BlockSpec(memory_space=pl.ANY),
        out_shape=jax.ShapeDtypeStruct((seq_len, n_embd), encoding.dtype),
        scratch_shapes=[
            pltpu.VMEM((2, _MAXCH, n_embd), jnp.float32),
            pltpu.SemaphoreType.DMA((len(_SIZES),)),
        ],
    )()


# speedup vs baseline: 1.0572x; 1.0572x over previous
import math

import jax
import jax.numpy as jnp
from jax.experimental import pallas as pl
from jax.experimental.pallas import tpu as pltpu


_LOG_BASE = math.log(10000.0)
_SEED = 16
_SIZES = (512, 1024, 1024, 1024, 512)
_MAXCH = max(_SIZES)


def _gen_all(o_ref, buf_ref, sem_ref):
    total, cols = o_ref.shape
    j = jax.lax.broadcasted_iota(jnp.int32, (1, cols), 1)
    k = (j // 2).astype(jnp.float32)
    w = jnp.exp(k * jnp.float32(-2.0 * _LOG_BASE / cols))
    phase = jnp.where(j % 2 == 1, jnp.float32(math.pi / 2), jnp.float32(0.0))
    r8 = jax.lax.broadcasted_iota(jnp.int32, (8, cols), 0)
    dmat = jnp.minimum(8 << r8, jnp.int32(_MAXCH // 2)).astype(jnp.float32)
    coefs = 2.0 * jnp.sin(dmat * w + jnp.float32(math.pi / 2))
    cof = {}
    d, ridx = 8, 0
    while d <= _MAXCH // 2:
        cof[d] = coefs[ridx:ridx + 1, :]
        ridx, d = ridx + 1, 2 * d
    r = jax.lax.broadcasted_iota(jnp.int32, (_SEED, cols), 0)
    seed_cur = jnp.sin(r.astype(jnp.float32) * w + phase)

    copies = []
    off = 0
    for c, size in enumerate(_SIZES):
        slot = c % 2
        if c >= 2:
            copies[c - 2][0].wait(); copies[c - 2][1].wait()
        buf_ref[slot, 0:_SEED, :] = seed_cur
        n = _SEED
        while n < size:
            d = n // 2
            coef = cof[d]
            prev_lo = buf_ref[slot, 0:d, :]
            prev_hi = buf_ref[slot, d:n, :]
            h1 = coef * prev_hi - prev_lo
            buf_ref[slot, n:n + d, :] = h1
            buf_ref[slot, n + d:2 * n, :] = coef * h1 - prev_hi
            n *= 2
        h2 = size // 2
        cp_a = pltpu.make_async_copy(
            buf_ref.at[slot, pl.ds(0, h2), :],
            o_ref.at[pl.ds(off, h2), :],
            sem_ref.at[c, 0],
        )
        cp_b = pltpu.make_async_copy(
            buf_ref.at[slot, pl.ds(h2, h2), :],
            o_ref.at[pl.ds(off + h2, h2), :],
            sem_ref.at[c, 1],
        )
        cp_a.start()
        cp_b.start()
        copies.append((cp_a, cp_b))
        off += size
        if c + 1 < len(_SIZES):
            h = size // 2
            seed_cur = (cof[h] * buf_ref[slot, h:h + _SEED, :]
                        - buf_ref[slot, 0:_SEED, :])
    for cp2 in copies[-2:]:
        cp2[0].wait()
        cp2[1].wait()


def kernel(x, encoding):
    seq_len = x.shape[1]
    n_embd = encoding.shape[1]
    return pl.pallas_call(
        _gen_all,
        out_specs=pl.BlockSpec(memory_space=pl.ANY),
        out_shape=jax.ShapeDtypeStruct((seq_len, n_embd), encoding.dtype),
        scratch_shapes=[
            pltpu.VMEM((2, _MAXCH, n_embd), jnp.float32),
            pltpu.SemaphoreType.DMA((len(_SIZES), 2)),
        ],
    )()
